# baseline (speedup 1.0000x reference)
import functools

import jax
import jax.numpy as jnp
from jax import lax
from jax.experimental import pallas as pl
from jax.experimental.pallas import tpu as pltpu
from jax.experimental.pallas import tpu_sc as plsc

_MARGIN = 0.5
_EPS = 1e-6


def _stage1_body(a_ref, p_ref, n_ref, in_ref, ip_ref, loss_ref,
                 sp_ref, sn_ref):
    a = a_ref[...]
    dp = a - p_ref[...] + _EPS
    dn = a - n_ref[...] + _EPS
    sp_ref[...] = jnp.sum(dp * dp, axis=-1)
    sn_ref[...] = jnp.sum(dn * dn, axis=-1)
    dist_pos = jnp.sqrt(sp_ref[...])
    dist_neg = jnp.sqrt(sn_ref[...])

    bb, nn = dist_pos.shape
    iota = lax.broadcasted_iota(jnp.int32, (bb, nn), 1)

    mask_neg = ((dist_pos - dist_neg) + _MARGIN) > 0
    neg_scores = jnp.where(mask_neg, dist_neg, -jnp.inf)
    mn = jnp.max(neg_scores, axis=-1, keepdims=True)
    idxn = jnp.min(jnp.where(neg_scores == mn, iota, nn), axis=-1)

    mask_pos = ((dist_neg - dist_pos) + _MARGIN) > 0
    pos_scores = jnp.where(mask_pos, dist_pos, jnp.inf)
    mp = jnp.min(pos_scores, axis=-1, keepdims=True)
    idxp = jnp.min(jnp.where(pos_scores == mp, iota, nn), axis=-1)

    in_ref[0, 0, :] = idxn
    ip_ref[0, 0, :] = idxp

    part = jnp.sum(jnp.maximum((_MARGIN + dist_pos) - dist_neg, 0.0))

    @pl.when(pl.program_id(0) == 0)
    def _():
        loss_ref[0, 0] = 0.0

    loss_ref[0, 0] += part


def _stage1(anchor, positive, negative, bb=128, interpret=False):
    B, N, D = anchor.shape
    grid = (B // bb,)
    in_spec = pl.BlockSpec((bb, N, D), lambda i: (i, 0, 0))
    idxn3, idxp3, loss_sum = pl.pallas_call(
        _stage1_body,
        grid=grid,
        in_specs=[in_spec, in_spec, in_spec],
        out_specs=[
            pl.BlockSpec((1, 1, bb), lambda i: (i, 0, 0)),
            pl.BlockSpec((1, 1, bb), lambda i: (i, 0, 0)),
            pl.BlockSpec(memory_space=pltpu.SMEM),
        ],
        out_shape=[
            jax.ShapeDtypeStruct((B // bb, 1, bb), jnp.int32),
            jax.ShapeDtypeStruct((B // bb, 1, bb), jnp.int32),
            jax.ShapeDtypeStruct((1, 1), jnp.float32),
        ],
        scratch_shapes=[
            pltpu.VMEM((bb, N), jnp.float32),
            pltpu.VMEM((bb, N), jnp.float32),
        ],
        interpret=interpret,
    )(anchor, positive, negative)
    return idxn3.reshape(B), idxp3.reshape(B), loss_sum[0, 0]


@functools.lru_cache(maxsize=None)
def _make_gather(B, N, D, K):
    info = plsc.get_sparse_core_info()
    NC, NS = info.num_cores, info.num_subcores
    NW = NC * NS
    BPW = B // NW
    NCHUNK = BPW // K
    mesh = plsc.VectorSubcoreMesh(core_axis_name="c", subcore_axis_name="s")

    @functools.partial(
        pl.kernel,
        mesh=mesh,
        out_type=[
            jax.ShapeDtypeStruct((B, N, D), jnp.float32),
            jax.ShapeDtypeStruct((B, N, D), jnp.float32),
        ],
        scratch_types=[
            pltpu.VMEM((NCHUNK, K), jnp.int32),
            pltpu.VMEM((NCHUNK, K), jnp.int32),
            pltpu.VMEM((K, N, D), jnp.float32),
            pltpu.VMEM((K, N, D), jnp.float32),
            pltpu.SemaphoreType.DMA,
            pltpu.SemaphoreType.DMA,
        ],
    )
    def _gather(neg_hbm, pos_hbm, ineg_hbm, ipos_hbm, hn_hbm, hp_hbm,
                ineg_v, ipos_v, buf_n, buf_p, sem_n, sem_p):
        wid = lax.axis_index("s") * NC + lax.axis_index("c")
        base = wid * BPW
        pltpu.sync_copy(ineg_hbm.at[pl.ds(wid * NCHUNK, NCHUNK)], ineg_v)
        pltpu.sync_copy(ipos_hbm.at[pl.ds(wid * NCHUNK, NCHUNK)], ipos_v)
        tabs = (neg_hbm, pos_hbm)
        idxs = (ineg_v, ipos_v)
        outs = (hn_hbm, hp_hbm)
        bufs = (buf_n, buf_p)
        sems = (sem_n, sem_p)

        def fire(t, c):
            pltpu.async_copy(tabs[t].at[idxs[t].at[c]], bufs[t], sems[t])

        def drain_scatter(t, c):
            pltpu.make_async_copy(tabs[t].at[pl.ds(0, K)], bufs[t],
                                  sems[t]).wait()
            pltpu.sync_copy(bufs[t], outs[t].at[pl.ds(base + c * K, K)])

        fire(0, 0)
        fire(1, 0)

        def body(c, _):
            drain_scatter(0, c - 1)
            fire(0, c)
            drain_scatter(1, c - 1)
            fire(1, c)
            return _

        lax.fori_loop(1, NCHUNK, body, None)
        drain_scatter(0, NCHUNK - 1)
        drain_scatter(1, NCHUNK - 1)

    return _gather


def kernel(anchor, positive, negative):
    B, N, D = anchor.shape
    idx_neg, idx_pos, loss_sum = _stage1(anchor, positive, negative)
    loss = loss_sum / jnp.float32(B * N)
    K = 2
    gather = _make_gather(B, N, D, K)
    hn, hp = gather(
        negative,
        positive,
        idx_neg.reshape(B // K, K),
        idx_pos.reshape(B // K, K),
    )
    return loss, hn, hp


# speedup vs baseline: 1.0271x; 1.0271x over previous
import functools

import jax
import jax.numpy as jnp
from jax import lax
from jax.experimental import pallas as pl
from jax.experimental.pallas import tpu as pltpu
from jax.experimental.pallas import tpu_sc as plsc

_MARGIN = 0.5
_EPS = 1e-6


def _stage1_body(a_ref, p_ref, n_ref, in_ref, ip_ref, loss_ref,
                 sp_ref, sn_ref):
    a = a_ref[...]
    dp = a - p_ref[...] + _EPS
    dn = a - n_ref[...] + _EPS
    sp_ref[...] = jnp.sum(dp * dp, axis=-1)
    sn_ref[...] = jnp.sum(dn * dn, axis=-1)
    dist_pos = jnp.sqrt(sp_ref[...])
    dist_neg = jnp.sqrt(sn_ref[...])

    bb, nn = dist_pos.shape
    iota = lax.broadcasted_iota(jnp.int32, (bb, nn), 1)

    mask_neg = ((dist_pos - dist_neg) + _MARGIN) > 0
    neg_scores = jnp.where(mask_neg, dist_neg, -jnp.inf)
    mn = jnp.max(neg_scores, axis=-1, keepdims=True)
    idxn = jnp.min(jnp.where(neg_scores == mn, iota, nn), axis=-1)

    mask_pos = ((dist_neg - dist_pos) + _MARGIN) > 0
    pos_scores = jnp.where(mask_pos, dist_pos, jnp.inf)
    mp = jnp.min(pos_scores, axis=-1, keepdims=True)
    idxp = jnp.min(jnp.where(pos_scores == mp, iota, nn), axis=-1)

    in_ref[0, 0, :] = idxn
    ip_ref[0, 0, :] = idxp

    part = jnp.sum(jnp.maximum((_MARGIN + dist_pos) - dist_neg, 0.0))

    @pl.when(pl.program_id(0) == 0)
    def _():
        loss_ref[0, 0] = 0.0

    loss_ref[0, 0] += part


def _stage1(anchor, positive, negative, bb=64, interpret=False):
    B, N, D = anchor.shape
    grid = (B // bb,)
    in_spec = pl.BlockSpec((bb, N, D), lambda i: (i, 0, 0))
    idxn3, idxp3, loss_sum = pl.pallas_call(
        _stage1_body,
        grid=grid,
        in_specs=[in_spec, in_spec, in_spec],
        out_specs=[
            pl.BlockSpec((1, 1, bb), lambda i: (i, 0, 0)),
            pl.BlockSpec((1, 1, bb), lambda i: (i, 0, 0)),
            pl.BlockSpec(memory_space=pltpu.SMEM),
        ],
        out_shape=[
            jax.ShapeDtypeStruct((B // bb, 1, bb), jnp.int32),
            jax.ShapeDtypeStruct((B // bb, 1, bb), jnp.int32),
            jax.ShapeDtypeStruct((1, 1), jnp.float32),
        ],
        scratch_shapes=[
            pltpu.VMEM((bb, N), jnp.float32),
            pltpu.VMEM((bb, N), jnp.float32),
        ],
        interpret=interpret,
    )(anchor, positive, negative)
    return idxn3.reshape(B), idxp3.reshape(B), loss_sum[0, 0]


@functools.lru_cache(maxsize=None)
def _make_gather(B, N, D, K):
    info = plsc.get_sparse_core_info()
    NC, NS = info.num_cores, info.num_subcores
    NW = NC * NS
    BPW = B // NW
    NCHUNK = BPW // K
    mesh = plsc.VectorSubcoreMesh(core_axis_name="c", subcore_axis_name="s")

    @functools.partial(
        pl.kernel,
        mesh=mesh,
        out_type=[
            jax.ShapeDtypeStruct((B, N, D), jnp.float32),
            jax.ShapeDtypeStruct((B, N, D), jnp.float32),
        ],
        scratch_types=[
            pltpu.VMEM((NCHUNK, K), jnp.int32),
            pltpu.VMEM((NCHUNK, K), jnp.int32),
            pltpu.VMEM((3, K, N, D), jnp.float32),
            pltpu.SemaphoreType.DMA,
            pltpu.SemaphoreType.DMA,
        ],
    )
    def _gather(neg_hbm, pos_hbm, ineg_hbm, ipos_hbm, hn_hbm, hp_hbm,
                ineg_v, ipos_v, bufs, sem_in, sem_out):
        wid = lax.axis_index("s") * NC + lax.axis_index("c")
        base = wid * BPW
        pltpu.sync_copy(ineg_hbm.at[pl.ds(wid * NCHUNK, NCHUNK)], ineg_v)
        pltpu.sync_copy(ipos_hbm.at[pl.ds(wid * NCHUNK, NCHUNK)], ipos_v)
        tabs = (neg_hbm, pos_hbm)
        idxs = (ineg_v, ipos_v)
        outs = (hn_hbm, hp_hbm)

        def fire_in(t, c, b):
            pltpu.async_copy(tabs[t].at[idxs[t].at[c]], bufs.at[b], sem_in)

        def wait_in(b):
            pltpu.make_async_copy(tabs[0].at[pl.ds(0, K)], bufs.at[b],
                                  sem_in).wait()

        def fire_out(t, c, b):
            pltpu.async_copy(bufs.at[b], outs[t].at[pl.ds(base + c * K, K)],
                             sem_out)

        def wait_out(b):
            pltpu.make_async_copy(tabs[0].at[pl.ds(0, K)], bufs.at[b],
                                  sem_out).wait()

        def step(o, u, with_guards):
            t, b = u % 2, u % 3
            c = 3 * o + u // 2
            free = functools.partial(wait_out, (u - 3) % 3)
            if u >= 3 or not with_guards:
                free()
            else:
                pl.when(o > 0)(free)
            fire_in(t, c, b)

            def drain():
                wait_in((u - 2) % 3)
                fire_out((u - 2) % 2, 3 * o + (u - 2) // 2, (u - 2) % 3)

            if u >= 2:
                drain()
            elif with_guards:
                pl.when(o > 0)(drain)
            else:
                drain()

        def body(o, _):
            for u in range(6):
                step(o, u, with_guards=True)
            return _

        n_outer = (2 * NCHUNK - 2) // 6
        lax.fori_loop(0, n_outer, body, None)
        for u in range(2):
            step(n_outer, u, with_guards=False)
        last = 2 * NCHUNK - 1
        for j in (last - 1, last):
            wait_in(j % 3)
            fire_out(j % 2, j // 2, j % 3)
        for j in (last - 2, last - 1, last):
            wait_out(j % 3)

    return _gather


def kernel(anchor, positive, negative):
    B, N, D = anchor.shape
    idx_neg, idx_pos, loss_sum = _stage1(anchor, positive, negative)
    loss = loss_sum / jnp.float32(B * N)
    K = 2
    gather = _make_gather(B, N, D, K)
    hn, hp = gather(
        negative,
        positive,
        idx_neg.reshape(B // K, K),
        idx_pos.reshape(B // K, K),
    )
    return loss, hn, hp
